# Initial kernel scaffold; baseline (speedup 1.0000x reference)
import jax
import jax.numpy as jnp
from jax.experimental import pallas as pl
from jax.experimental.pallas import tpu as pltpu

_B = 4096
_E = 64
_C = 16
_K = 25
_LOSS_WEIGHT = 0.5
_MIN_CONF = 0.25
_BLK = 256


def _stats_kernel(cat_ref, enc_ref, onehot_ref, mg_ref, sq_ref,
                  conf_sum_ref, nclust_ref):
    cat = cat_ref[...]
    m = jnp.max(cat, axis=1, keepdims=True)
    lane = jax.lax.broadcasted_iota(jnp.int32, cat.shape, 1)
    first = jnp.min(jnp.where(cat == m, lane, _C), axis=1, keepdims=True)
    onehot = (lane == first).astype(jnp.float32)
    onehot_ref[...] = onehot
    mg_ref[...] = m
    conf = (m >= _MIN_CONF).astype(jnp.float32)
    ccounts = jnp.sum(onehot * conf, axis=0, keepdims=True)
    nclust_ref[0, 0] = jnp.sum((ccounts > 0).astype(jnp.float32))
    conf_sum_ref[0, 0] = jnp.sum(m)
    enc = enc_ref[...]
    sq_ref[...] = jnp.sum(enc * enc, axis=1, keepdims=True)


def _entropy_kernel(enc_blk_ref, encT_ref, sq_row_ref, sq_blk_ref,
                    onehot_ref, mg_blk_ref, ent_ref, entsum_ref):
    x = enc_blk_ref[...]
    xt = encT_ref[...]
    mm = jax.lax.dot_general(
        x, xt, (((1,), (0,)), ((), ())),
        preferred_element_type=jnp.float32,
        precision=jax.lax.Precision.HIGHEST)
    d2 = sq_blk_ref[...] + sq_row_ref[...] - 2.0 * mm
    d2 = jnp.maximum(d2, 0.0)
    bits = jax.lax.bitcast_convert_type(d2, jnp.int32)
    hi0 = jnp.max(bits, axis=1, keepdims=True)
    lo0 = jnp.zeros_like(hi0)

    def body(_, carry):
        lo, hi = carry
        mid = lo + (hi - lo) // 2
        cnt = jnp.sum((bits <= mid).astype(jnp.int32), axis=1,
                      keepdims=True)
        take = cnt >= (_K + 1)
        return jnp.where(take, lo, mid + 1), jnp.where(take, mid, hi)

    _, v = jax.lax.fori_loop(0, 31, body, (lo0, hi0))
    mask = (bits < v).astype(jnp.float32)
    counts = jax.lax.dot_general(
        mask, onehot_ref[...], (((1,), (0,)), ((), ())),
        preferred_element_type=jnp.float32)
    totals = jnp.sum(mask, axis=1, keepdims=True)
    bins = counts / totals
    purity = -jnp.sum(bins * jnp.log(bins + 1e-5), axis=1,
                      keepdims=True)
    ent = purity * mg_blk_ref[...]
    ent_ref[...] = ent

    @pl.when(pl.program_id(0) == 0)
    def _init():
        entsum_ref[0, 0] = 0.0

    entsum_ref[0, 0] += jnp.sum(ent)


def kernel(encodings, categorical):
    onehot, mg, sq, conf_sum, nclust = pl.pallas_call(
        _stats_kernel,
        out_shape=[
            jax.ShapeDtypeStruct((_B, _C), jnp.float32),
            jax.ShapeDtypeStruct((_B, 1), jnp.float32),
            jax.ShapeDtypeStruct((_B, 1), jnp.float32),
            jax.ShapeDtypeStruct((1, 1), jnp.float32),
            jax.ShapeDtypeStruct((1, 1), jnp.float32),
        ],
    )(categorical, encodings)

    ent, entsum = pl.pallas_call(
        _entropy_kernel,
        grid=(_B // _BLK,),
        in_specs=[
            pl.BlockSpec((_BLK, _E), lambda i: (i, 0)),
            pl.BlockSpec((_E, _B), lambda i: (0, 0)),
            pl.BlockSpec((1, _B), lambda i: (0, 0)),
            pl.BlockSpec((_BLK, 1), lambda i: (i, 0)),
            pl.BlockSpec((_B, _C), lambda i: (0, 0)),
            pl.BlockSpec((_BLK, 1), lambda i: (i, 0)),
        ],
        out_specs=[
            pl.BlockSpec((_BLK, 1), lambda i: (i, 0)),
            pl.BlockSpec((1, 1), lambda i: (0, 0)),
        ],
        out_shape=[
            jax.ShapeDtypeStruct((_B, 1), jnp.float32),
            jax.ShapeDtypeStruct((1, 1), jnp.float32),
        ],
        compiler_params=pltpu.CompilerParams(
            dimension_semantics=("arbitrary",)),
    )(encodings, encodings.T, sq.T, sq, onehot, mg)

    neighbourhood_entropy = ent[:, 0]
    number_of_clusters = nclust[0, 0]
    average_confidence = conf_sum[0, 0] / _B
    average_neigh_entropy = entsum[0, 0] / _B
    loss = _LOSS_WEIGHT * average_neigh_entropy
    return (encodings, neighbourhood_entropy, number_of_clusters,
            average_confidence, average_neigh_entropy, loss)


# speedup vs baseline: 10.6821x; 10.6821x over previous
import jax
import jax.numpy as jnp
from jax.experimental import pallas as pl
from jax.experimental.pallas import tpu as pltpu

_B = 4096
_E = 64
_C = 16
_K = 25
_LOSS_WEIGHT = 0.5
_MIN_CONF = 0.25
_BLK = 256


def _stats_kernel(cat_ref, enc_ref, onehot_ref, mg_ref, sq_ref,
                  conf_sum_ref, nclust_ref):
    cat = cat_ref[...]
    m = jnp.max(cat, axis=1, keepdims=True)
    lane = jax.lax.broadcasted_iota(jnp.int32, cat.shape, 1)
    first = jnp.min(jnp.where(cat == m, lane, _C), axis=1, keepdims=True)
    onehot = (lane == first).astype(jnp.float32)
    onehot_ref[...] = onehot
    mg_ref[...] = m
    conf = (m >= _MIN_CONF).astype(jnp.float32)
    ccounts = jnp.sum(onehot * conf, axis=0, keepdims=True)
    nclust_ref[...] = jnp.sum((ccounts > 0).astype(jnp.float32), axis=1,
                              keepdims=True)
    conf_sum_ref[...] = jnp.sum(m, axis=(0, 1), keepdims=True)
    enc = enc_ref[...]
    sq_ref[...] = jnp.sum(enc * enc, axis=1, keepdims=True)


def _entropy_kernel(enc_blk_ref, encT_ref, sq_row_ref, sq_blk_ref,
                    onehot_ref, mg_blk_ref, ent_ref, entsum_ref):
    x = enc_blk_ref[...]
    xt = encT_ref[...]
    mm = jax.lax.dot_general(
        x, xt, (((1,), (0,)), ((), ())),
        preferred_element_type=jnp.float32,
        precision=jax.lax.Precision.HIGHEST)
    d2 = sq_blk_ref[...] + sq_row_ref[...] - 2.0 * mm
    d2 = jnp.maximum(d2, 0.0)
    bits = jax.lax.bitcast_convert_type(d2, jnp.int32)
    hi0 = jnp.max(bits, axis=1, keepdims=True)
    lo0 = jnp.zeros_like(hi0)

    def body(_, carry):
        lo, hi = carry
        mid = lo + (hi - lo) // 2
        cnt = jnp.sum((bits <= mid).astype(jnp.int32), axis=1,
                      keepdims=True)
        take = cnt >= (_K + 1)
        return jnp.where(take, lo, mid + 1), jnp.where(take, mid, hi)

    _, v = jax.lax.fori_loop(0, 31, body, (lo0, hi0))
    mask = (bits < v).astype(jnp.float32)
    counts = jax.lax.dot_general(
        mask, onehot_ref[...], (((1,), (0,)), ((), ())),
        preferred_element_type=jnp.float32)
    totals = jnp.sum(mask, axis=1, keepdims=True)
    bins = counts / totals
    purity = -jnp.sum(bins * jnp.log(bins + 1e-5), axis=1,
                      keepdims=True)
    ent = purity * mg_blk_ref[...]
    ent_ref[...] = ent

    @pl.when(pl.program_id(0) == 0)
    def _init():
        entsum_ref[...] = jnp.zeros((1, 1), jnp.float32)

    entsum_ref[...] += jnp.sum(ent, axis=(0, 1), keepdims=True)


def kernel(encodings, categorical):
    onehot, mg, sq, conf_sum, nclust = pl.pallas_call(
        _stats_kernel,
        out_shape=[
            jax.ShapeDtypeStruct((_B, _C), jnp.float32),
            jax.ShapeDtypeStruct((_B, 1), jnp.float32),
            jax.ShapeDtypeStruct((_B, 1), jnp.float32),
            jax.ShapeDtypeStruct((1, 1), jnp.float32),
            jax.ShapeDtypeStruct((1, 1), jnp.float32),
        ],
    )(categorical, encodings)

    ent, entsum = pl.pallas_call(
        _entropy_kernel,
        grid=(_B // _BLK,),
        in_specs=[
            pl.BlockSpec((_BLK, _E), lambda i: (i, 0)),
            pl.BlockSpec((_E, _B), lambda i: (0, 0)),
            pl.BlockSpec((1, _B), lambda i: (0, 0)),
            pl.BlockSpec((_BLK, 1), lambda i: (i, 0)),
            pl.BlockSpec((_B, _C), lambda i: (0, 0)),
            pl.BlockSpec((_BLK, 1), lambda i: (i, 0)),
        ],
        out_specs=[
            pl.BlockSpec((_BLK, 1), lambda i: (i, 0)),
            pl.BlockSpec((1, 1), lambda i: (0, 0)),
        ],
        out_shape=[
            jax.ShapeDtypeStruct((_B, 1), jnp.float32),
            jax.ShapeDtypeStruct((1, 1), jnp.float32),
        ],
        compiler_params=pltpu.CompilerParams(
            dimension_semantics=("arbitrary",)),
    )(encodings, encodings.T, sq.T, sq, onehot, mg)

    neighbourhood_entropy = ent[:, 0]
    number_of_clusters = nclust[0, 0]
    average_confidence = conf_sum[0, 0] / _B
    average_neigh_entropy = entsum[0, 0] / _B
    loss = _LOSS_WEIGHT * average_neigh_entropy
    return (encodings, neighbourhood_entropy, number_of_clusters,
            average_confidence, average_neigh_entropy, loss)


# speedup vs baseline: 13.5488x; 1.2684x over previous
import jax
import jax.numpy as jnp
from jax.experimental import pallas as pl
from jax.experimental.pallas import tpu as pltpu

_B = 4096
_E = 64
_C = 16
_K = 25
_LOSS_WEIGHT = 0.5
_MIN_CONF = 0.25
_BLK = 256


def _stats_kernel(cat_ref, enc_ref, onehot_ref, mg_ref, sq_ref,
                  conf_sum_ref, nclust_ref):
    cat = cat_ref[...]
    m = jnp.max(cat, axis=1, keepdims=True)
    lane = jax.lax.broadcasted_iota(jnp.int32, cat.shape, 1)
    first = jnp.min(jnp.where(cat == m, lane, _C), axis=1, keepdims=True)
    onehot = (lane == first).astype(jnp.float32)
    onehot_ref[...] = onehot.astype(jnp.bfloat16)
    mg_ref[...] = m
    conf = (m >= _MIN_CONF).astype(jnp.float32)
    ccounts = jnp.sum(onehot * conf, axis=0, keepdims=True)
    nclust_ref[...] = jnp.sum((ccounts > 0).astype(jnp.float32), axis=1,
                              keepdims=True)
    conf_sum_ref[...] = jnp.sum(m, axis=(0, 1), keepdims=True)
    enc = enc_ref[...]
    sq_ref[...] = jnp.sum(enc * enc, axis=1, keepdims=True)


def _count_le(arr16, mid):
    mid16 = mid.astype(jnp.int16)
    nchunks = arr16.shape[1] // 128
    acc = jnp.zeros((arr16.shape[0], 128), jnp.int16)
    for t in range(nchunks):
        acc = acc + (arr16[:, t * 128:(t + 1) * 128]
                     <= mid16).astype(jnp.int16)
    return jnp.sum(acc.astype(jnp.int32), axis=1, keepdims=True)


def _entropy_kernel(enc_blk_ref, encT_ref, sq_row_ref, sq_blk_ref,
                    onehot_ref, mg_blk_ref, ent_ref, entsum_ref):
    x = enc_blk_ref[...]
    xt = encT_ref[...]
    mm = jax.lax.dot_general(
        x, xt, (((1,), (0,)), ((), ())),
        preferred_element_type=jnp.float32,
        precision=jax.lax.Precision.HIGHEST)
    d2 = sq_blk_ref[...] + sq_row_ref[...] - 2.0 * mm
    d2 = jnp.maximum(d2, 0.0)
    bits = jax.lax.bitcast_convert_type(d2, jnp.int32)

    top = (bits >> 16).astype(jnp.int16)
    loA = jnp.zeros((_BLK, 1), jnp.int32)
    hiA = jnp.full((_BLK, 1), 32767, jnp.int32)

    def body_a(_, carry):
        lo, hi = carry
        mid = lo + (hi - lo) // 2
        cnt = _count_le(top, mid)
        take = cnt >= (_K + 1)
        return jnp.where(take, lo, mid + 1), jnp.where(take, mid, hi)

    _, t_hi = jax.lax.fori_loop(0, 15, body_a, (loA, hiA))
    t16 = t_hi.astype(jnp.int16)

    c0 = _count_le(top, t_hi - 1)
    rank = (_K + 1) - c0

    klow = ((bits ^ 0x8000) & 0xFFFF).astype(jnp.int16)
    key = jnp.where(top == t16, klow, jnp.int16(0x7FFF))
    loB = jnp.full((_BLK, 1), -32768, jnp.int32)
    hiB = jnp.full((_BLK, 1), 32767, jnp.int32)

    def body_b(_, carry):
        lo, hi = carry
        mid = lo + (hi - lo) // 2
        cnt = _count_le(key, mid)
        take = cnt >= rank
        return jnp.where(take, lo, mid + 1), jnp.where(take, mid, hi)

    _, k_hi = jax.lax.fori_loop(0, 16, body_b, (loB, hiB))
    vbits = (t_hi << 16) | ((k_hi & 0xFFFF) ^ 0x8000)

    mask = (bits < vbits).astype(jnp.float32).astype(
        jnp.bfloat16)
    counts = jax.lax.dot_general(
        mask, onehot_ref[...], (((1,), (0,)), ((), ())),
        preferred_element_type=jnp.float32)
    totals = jnp.sum(counts, axis=1, keepdims=True)
    bins = counts / totals
    purity = -jnp.sum(bins * jnp.log(bins + 1e-5), axis=1,
                      keepdims=True)
    ent = purity * mg_blk_ref[...]
    ent_ref[...] = ent

    @pl.when(pl.program_id(0) == 0)
    def _init():
        entsum_ref[...] = jnp.zeros((1, 1), jnp.float32)

    entsum_ref[...] += jnp.sum(ent, axis=(0, 1), keepdims=True)


def kernel(encodings, categorical):
    onehot, mg, sq, conf_sum, nclust = pl.pallas_call(
        _stats_kernel,
        out_shape=[
            jax.ShapeDtypeStruct((_B, _C), jnp.bfloat16),
            jax.ShapeDtypeStruct((_B, 1), jnp.float32),
            jax.ShapeDtypeStruct((_B, 1), jnp.float32),
            jax.ShapeDtypeStruct((1, 1), jnp.float32),
            jax.ShapeDtypeStruct((1, 1), jnp.float32),
        ],
    )(categorical, encodings)

    ent, entsum = pl.pallas_call(
        _entropy_kernel,
        grid=(_B // _BLK,),
        in_specs=[
            pl.BlockSpec((_BLK, _E), lambda i: (i, 0)),
            pl.BlockSpec((_E, _B), lambda i: (0, 0)),
            pl.BlockSpec((1, _B), lambda i: (0, 0)),
            pl.BlockSpec((_BLK, 1), lambda i: (i, 0)),
            pl.BlockSpec((_B, _C), lambda i: (0, 0)),
            pl.BlockSpec((_BLK, 1), lambda i: (i, 0)),
        ],
        out_specs=[
            pl.BlockSpec((_BLK, 1), lambda i: (i, 0)),
            pl.BlockSpec((1, 1), lambda i: (0, 0)),
        ],
        out_shape=[
            jax.ShapeDtypeStruct((_B, 1), jnp.float32),
            jax.ShapeDtypeStruct((1, 1), jnp.float32),
        ],
        compiler_params=pltpu.CompilerParams(
            dimension_semantics=("arbitrary",)),
    )(encodings, encodings.T, sq.T, sq, onehot, mg)

    neighbourhood_entropy = ent[:, 0]
    number_of_clusters = nclust[0, 0]
    average_confidence = conf_sum[0, 0] / _B
    average_neigh_entropy = entsum[0, 0] / _B
    loss = _LOSS_WEIGHT * average_neigh_entropy
    return (encodings, neighbourhood_entropy, number_of_clusters,
            average_confidence, average_neigh_entropy, loss)


# speedup vs baseline: 15.2728x; 1.1272x over previous
import jax
import jax.numpy as jnp
from jax.experimental import pallas as pl
from jax.experimental.pallas import tpu as pltpu

_B = 4096
_E = 64
_C = 16
_K = 25
_LOSS_WEIGHT = 0.5
_MIN_CONF = 0.25
_BLK = 256


def _stats_kernel(cat_ref, enc_ref, onehot_ref, mg_ref, sq_ref,
                  conf_sum_ref, nclust_ref):
    cat = cat_ref[...]
    m = jnp.max(cat, axis=1, keepdims=True)
    lane = jax.lax.broadcasted_iota(jnp.int32, cat.shape, 1)
    first = jnp.min(jnp.where(cat == m, lane, _C), axis=1, keepdims=True)
    onehot = (lane == first).astype(jnp.float32)
    onehot_ref[...] = onehot.astype(jnp.bfloat16)
    mg_ref[...] = m
    conf = (m >= _MIN_CONF).astype(jnp.float32)
    ccounts = jnp.sum(onehot * conf, axis=0, keepdims=True)
    nclust_ref[...] = jnp.sum((ccounts > 0).astype(jnp.float32), axis=1,
                              keepdims=True)
    conf_sum_ref[...] = jnp.sum(m, axis=(0, 1), keepdims=True)
    enc = enc_ref[...]
    sq_ref[...] = jnp.sum(enc * enc, axis=1, keepdims=True)


def _count_le(arr16, mid):
    mid16 = mid.astype(jnp.int16)
    nchunks = arr16.shape[1] // 128
    acc = jnp.zeros((arr16.shape[0], 128), jnp.int16)
    for t in range(nchunks):
        acc = acc + (arr16[:, t * 128:(t + 1) * 128]
                     <= mid16).astype(jnp.int16)
    return jnp.sum(acc.astype(jnp.int32), axis=1, keepdims=True)


def _entropy_kernel(enc_blk_ref, encT_ref, sq_row_ref, sq_blk_ref,
                    onehot_ref, mg_blk_ref, ent_ref, entsum_ref):
    x = enc_blk_ref[...]
    xt = encT_ref[...]
    mm = jax.lax.dot_general(
        x, xt, (((1,), (0,)), ((), ())),
        preferred_element_type=jnp.float32,
        precision=jax.lax.Precision.DEFAULT)
    d2 = sq_blk_ref[...] + sq_row_ref[...] - 2.0 * mm
    d2 = jnp.maximum(d2, 0.0)
    bits = jax.lax.bitcast_convert_type(d2, jnp.int32)

    top = (bits >> 16).astype(jnp.int16)
    loA = jnp.zeros((_BLK, 1), jnp.int32)
    hiA = jnp.full((_BLK, 1), 32767, jnp.int32)

    def body_a(_, carry):
        lo, hi = carry
        mid = lo + (hi - lo) // 2
        cnt = _count_le(top, mid)
        take = cnt >= (_K + 1)
        return jnp.where(take, lo, mid + 1), jnp.where(take, mid, hi)

    _, t_hi = jax.lax.fori_loop(0, 15, body_a, (loA, hiA))
    t16 = t_hi.astype(jnp.int16)

    c0 = _count_le(top, t_hi - 1)
    rank = (_K + 1) - c0

    klow = ((bits ^ 0x8000) & 0xFFFF).astype(jnp.int16)
    key = jnp.where(top == t16, klow, jnp.int16(0x7FFF))
    loB = jnp.full((_BLK, 1), -32768, jnp.int32)
    hiB = jnp.full((_BLK, 1), 32767, jnp.int32)

    def body_b(_, carry):
        lo, hi = carry
        mid = lo + (hi - lo) // 2
        cnt = _count_le(key, mid)
        take = cnt >= rank
        return jnp.where(take, lo, mid + 1), jnp.where(take, mid, hi)

    _, k_hi = jax.lax.fori_loop(0, 16, body_b, (loB, hiB))
    vbits = (t_hi << 16) | ((k_hi & 0xFFFF) ^ 0x8000)

    mask = (bits < vbits).astype(jnp.float32).astype(
        jnp.bfloat16)
    counts = jax.lax.dot_general(
        mask, onehot_ref[...], (((1,), (0,)), ((), ())),
        preferred_element_type=jnp.float32)
    totals = jnp.sum(counts, axis=1, keepdims=True)
    bins = counts / totals
    purity = -jnp.sum(bins * jnp.log(bins + 1e-5), axis=1,
                      keepdims=True)
    ent = purity * mg_blk_ref[...]
    ent_ref[...] = ent

    @pl.when(pl.program_id(0) == 0)
    def _init():
        entsum_ref[...] = jnp.zeros((1, 1), jnp.float32)

    entsum_ref[...] += jnp.sum(ent, axis=(0, 1), keepdims=True)


def kernel(encodings, categorical):
    onehot, mg, sq, conf_sum, nclust = pl.pallas_call(
        _stats_kernel,
        out_shape=[
            jax.ShapeDtypeStruct((_B, _C), jnp.bfloat16),
            jax.ShapeDtypeStruct((_B, 1), jnp.float32),
            jax.ShapeDtypeStruct((_B, 1), jnp.float32),
            jax.ShapeDtypeStruct((1, 1), jnp.float32),
            jax.ShapeDtypeStruct((1, 1), jnp.float32),
        ],
    )(categorical, encodings)

    ent, entsum = pl.pallas_call(
        _entropy_kernel,
        grid=(_B // _BLK,),
        in_specs=[
            pl.BlockSpec((_BLK, _E), lambda i: (i, 0)),
            pl.BlockSpec((_E, _B), lambda i: (0, 0)),
            pl.BlockSpec((1, _B), lambda i: (0, 0)),
            pl.BlockSpec((_BLK, 1), lambda i: (i, 0)),
            pl.BlockSpec((_B, _C), lambda i: (0, 0)),
            pl.BlockSpec((_BLK, 1), lambda i: (i, 0)),
        ],
        out_specs=[
            pl.BlockSpec((_BLK, 1), lambda i: (i, 0)),
            pl.BlockSpec((1, 1), lambda i: (0, 0)),
        ],
        out_shape=[
            jax.ShapeDtypeStruct((_B, 1), jnp.float32),
            jax.ShapeDtypeStruct((1, 1), jnp.float32),
        ],
        compiler_params=pltpu.CompilerParams(
            dimension_semantics=("arbitrary",)),
    )(encodings, encodings.T, sq.T, sq, onehot, mg)

    neighbourhood_entropy = ent[:, 0]
    number_of_clusters = nclust[0, 0]
    average_confidence = conf_sum[0, 0] / _B
    average_neigh_entropy = entsum[0, 0] / _B
    loss = _LOSS_WEIGHT * average_neigh_entropy
    return (encodings, neighbourhood_entropy, number_of_clusters,
            average_confidence, average_neigh_entropy, loss)


# speedup vs baseline: 16.0678x; 1.0521x over previous
import jax
import jax.numpy as jnp
from jax.experimental import pallas as pl
from jax.experimental.pallas import tpu as pltpu

_B = 4096
_E = 64
_C = 16
_K = 25
_LOSS_WEIGHT = 0.5
_MIN_CONF = 0.25
_BLK = 512


def _stats_kernel(cat_ref, enc_ref, onehot_ref, mg_ref, sq_ref,
                  conf_sum_ref, nclust_ref):
    cat = cat_ref[...]
    m = jnp.max(cat, axis=1, keepdims=True)
    lane = jax.lax.broadcasted_iota(jnp.int32, cat.shape, 1)
    first = jnp.min(jnp.where(cat == m, lane, _C), axis=1, keepdims=True)
    onehot = (lane == first).astype(jnp.float32)
    onehot_ref[...] = onehot.astype(jnp.bfloat16)
    mg_ref[...] = m
    conf = (m >= _MIN_CONF).astype(jnp.float32)
    ccounts = jnp.sum(onehot * conf, axis=0, keepdims=True)
    nclust_ref[...] = jnp.sum((ccounts > 0).astype(jnp.float32), axis=1,
                              keepdims=True)
    conf_sum_ref[...] = jnp.sum(m, axis=(0, 1), keepdims=True)
    enc = enc_ref[...]
    sq_ref[...] = jnp.sum(enc * enc, axis=1, keepdims=True)


def _count_le(arr16, mid):
    mid16 = mid.astype(jnp.int16)
    nchunks = arr16.shape[1] // 128
    acc = jnp.zeros((arr16.shape[0], 128), jnp.int16)
    for t in range(nchunks):
        acc = acc + (arr16[:, t * 128:(t + 1) * 128]
                     <= mid16).astype(jnp.int16)
    return jnp.sum(acc.astype(jnp.int32), axis=1, keepdims=True)


def _entropy_kernel(enc_blk_ref, encT_ref, sq_row_ref, sq_blk_ref,
                    onehot_ref, mg_blk_ref, ent_ref, entsum_ref):
    x = enc_blk_ref[...]
    xt = encT_ref[...]
    mm = jax.lax.dot_general(
        x, xt, (((1,), (0,)), ((), ())),
        preferred_element_type=jnp.float32,
        precision=jax.lax.Precision.DEFAULT)
    d2 = sq_blk_ref[...] + sq_row_ref[...] - 2.0 * mm
    d2 = jnp.maximum(d2, 0.0)
    bits = jax.lax.bitcast_convert_type(d2, jnp.int32)

    top = (bits >> 16).astype(jnp.int16)
    loA = jnp.zeros((_BLK, 1), jnp.int32)
    hiA = jnp.full((_BLK, 1), 32767, jnp.int32)

    def body_a(_, carry):
        lo, hi = carry
        mid = lo + (hi - lo) // 2
        cnt = _count_le(top, mid)
        take = cnt >= (_K + 1)
        return jnp.where(take, lo, mid + 1), jnp.where(take, mid, hi)

    _, t_hi = jax.lax.fori_loop(0, 15, body_a, (loA, hiA))
    t16 = t_hi.astype(jnp.int16)

    c0 = _count_le(top, t_hi - 1)
    rank = (_K + 1) - c0

    klow = ((bits ^ 0x8000) & 0xFFFF).astype(jnp.int16)
    key = jnp.where(top == t16, klow, jnp.int16(0x7FFF))
    loB = jnp.full((_BLK, 1), -32768, jnp.int32)
    hiB = jnp.full((_BLK, 1), 32767, jnp.int32)

    def body_b(_, carry):
        lo, hi = carry
        mid = lo + (hi - lo) // 2
        cnt = _count_le(key, mid)
        take = cnt >= rank
        return jnp.where(take, lo, mid + 1), jnp.where(take, mid, hi)

    _, k_hi = jax.lax.fori_loop(0, 16, body_b, (loB, hiB))
    vbits = (t_hi << 16) | ((k_hi & 0xFFFF) ^ 0x8000)

    mask = (bits < vbits).astype(jnp.float32).astype(
        jnp.bfloat16)
    counts = jax.lax.dot_general(
        mask, onehot_ref[...], (((1,), (0,)), ((), ())),
        preferred_element_type=jnp.float32)
    totals = jnp.sum(counts, axis=1, keepdims=True)
    bins = counts / totals
    purity = -jnp.sum(bins * jnp.log(bins + 1e-5), axis=1,
                      keepdims=True)
    ent = purity * mg_blk_ref[...]
    ent_ref[...] = ent

    @pl.when(pl.program_id(0) == 0)
    def _init():
        entsum_ref[...] = jnp.zeros((1, 1), jnp.float32)

    entsum_ref[...] += jnp.sum(ent, axis=(0, 1), keepdims=True)


def kernel(encodings, categorical):
    onehot, mg, sq, conf_sum, nclust = pl.pallas_call(
        _stats_kernel,
        out_shape=[
            jax.ShapeDtypeStruct((_B, _C), jnp.bfloat16),
            jax.ShapeDtypeStruct((_B, 1), jnp.float32),
            jax.ShapeDtypeStruct((_B, 1), jnp.float32),
            jax.ShapeDtypeStruct((1, 1), jnp.float32),
            jax.ShapeDtypeStruct((1, 1), jnp.float32),
        ],
    )(categorical, encodings)

    ent, entsum = pl.pallas_call(
        _entropy_kernel,
        grid=(_B // _BLK,),
        in_specs=[
            pl.BlockSpec((_BLK, _E), lambda i: (i, 0)),
            pl.BlockSpec((_E, _B), lambda i: (0, 0)),
            pl.BlockSpec((1, _B), lambda i: (0, 0)),
            pl.BlockSpec((_BLK, 1), lambda i: (i, 0)),
            pl.BlockSpec((_B, _C), lambda i: (0, 0)),
            pl.BlockSpec((_BLK, 1), lambda i: (i, 0)),
        ],
        out_specs=[
            pl.BlockSpec((_BLK, 1), lambda i: (i, 0)),
            pl.BlockSpec((1, 1), lambda i: (0, 0)),
        ],
        out_shape=[
            jax.ShapeDtypeStruct((_B, 1), jnp.float32),
            jax.ShapeDtypeStruct((1, 1), jnp.float32),
        ],
        compiler_params=pltpu.CompilerParams(
            dimension_semantics=("arbitrary",)),
    )(encodings, encodings.T, sq.T, sq, onehot, mg)

    neighbourhood_entropy = ent[:, 0]
    number_of_clusters = nclust[0, 0]
    average_confidence = conf_sum[0, 0] / _B
    average_neigh_entropy = entsum[0, 0] / _B
    loss = _LOSS_WEIGHT * average_neigh_entropy
    return (encodings, neighbourhood_entropy, number_of_clusters,
            average_confidence, average_neigh_entropy, loss)


# speedup vs baseline: 16.5275x; 1.0286x over previous
import jax
import jax.numpy as jnp
from jax.experimental import pallas as pl
from jax.experimental.pallas import tpu as pltpu

_B = 4096
_E = 64
_C = 16
_K = 25
_LOSS_WEIGHT = 0.5
_MIN_CONF = 0.25
_BLK = 1024


def _stats_kernel(cat_ref, enc_ref, onehot_ref, mg_ref, sq_ref,
                  conf_sum_ref, nclust_ref):
    cat = cat_ref[...]
    m = jnp.max(cat, axis=1, keepdims=True)
    lane = jax.lax.broadcasted_iota(jnp.int32, cat.shape, 1)
    first = jnp.min(jnp.where(cat == m, lane, _C), axis=1, keepdims=True)
    onehot = (lane == first).astype(jnp.float32)
    onehot_ref[...] = onehot.astype(jnp.bfloat16)
    mg_ref[...] = m
    conf = (m >= _MIN_CONF).astype(jnp.float32)
    ccounts = jnp.sum(onehot * conf, axis=0, keepdims=True)
    nclust_ref[...] = jnp.sum((ccounts > 0).astype(jnp.float32), axis=1,
                              keepdims=True)
    conf_sum_ref[...] = jnp.sum(m, axis=(0, 1), keepdims=True)
    enc = enc_ref[...]
    sq_ref[...] = jnp.sum(enc * enc, axis=1, keepdims=True)


def _count_le(arr16, mid):
    mid16 = mid.astype(jnp.int16)
    nchunks = arr16.shape[1] // 128
    acc = jnp.zeros((arr16.shape[0], 128), jnp.int16)
    for t in range(nchunks):
        acc = acc + (arr16[:, t * 128:(t + 1) * 128]
                     <= mid16).astype(jnp.int16)
    return jnp.sum(acc.astype(jnp.int32), axis=1, keepdims=True)


def _entropy_kernel(enc_blk_ref, encT_ref, sq_row_ref, sq_blk_ref,
                    onehot_ref, mg_blk_ref, ent_ref, entsum_ref):
    x = enc_blk_ref[...]
    xt = encT_ref[...]
    mm = jax.lax.dot_general(
        x, xt, (((1,), (0,)), ((), ())),
        preferred_element_type=jnp.float32,
        precision=jax.lax.Precision.DEFAULT)
    d2 = sq_blk_ref[...] + sq_row_ref[...] - 2.0 * mm
    d2 = jnp.maximum(d2, 0.0)
    bits = jax.lax.bitcast_convert_type(d2, jnp.int32)

    top = (bits >> 16).astype(jnp.int16)
    loA = jnp.zeros((_BLK, 1), jnp.int32)
    hiA = jnp.full((_BLK, 1), 32767, jnp.int32)

    def body_a(_, carry):
        lo, hi = carry
        mid = lo + (hi - lo) // 2
        cnt = _count_le(top, mid)
        take = cnt >= (_K + 1)
        return jnp.where(take, lo, mid + 1), jnp.where(take, mid, hi)

    _, t_hi = jax.lax.fori_loop(0, 15, body_a, (loA, hiA))
    t16 = t_hi.astype(jnp.int16)

    c0 = _count_le(top, t_hi - 1)
    rank = (_K + 1) - c0

    klow = ((bits ^ 0x8000) & 0xFFFF).astype(jnp.int16)
    key = jnp.where(top == t16, klow, jnp.int16(0x7FFF))
    loB = jnp.full((_BLK, 1), -32768, jnp.int32)
    hiB = jnp.full((_BLK, 1), 32767, jnp.int32)

    def body_b(_, carry):
        lo, hi = carry
        mid = lo + (hi - lo) // 2
        cnt = _count_le(key, mid)
        take = cnt >= rank
        return jnp.where(take, lo, mid + 1), jnp.where(take, mid, hi)

    _, k_hi = jax.lax.fori_loop(0, 16, body_b, (loB, hiB))
    vbits = (t_hi << 16) | ((k_hi & 0xFFFF) ^ 0x8000)

    mask = (bits < vbits).astype(jnp.float32).astype(
        jnp.bfloat16)
    counts = jax.lax.dot_general(
        mask, onehot_ref[...], (((1,), (0,)), ((), ())),
        preferred_element_type=jnp.float32)
    totals = jnp.sum(counts, axis=1, keepdims=True)
    bins = counts / totals
    purity = -jnp.sum(bins * jnp.log(bins + 1e-5), axis=1,
                      keepdims=True)
    ent = purity * mg_blk_ref[...]
    ent_ref[...] = ent

    @pl.when(pl.program_id(0) == 0)
    def _init():
        entsum_ref[...] = jnp.zeros((1, 1), jnp.float32)

    entsum_ref[...] += jnp.sum(ent, axis=(0, 1), keepdims=True)


def kernel(encodings, categorical):
    onehot, mg, sq, conf_sum, nclust = pl.pallas_call(
        _stats_kernel,
        out_shape=[
            jax.ShapeDtypeStruct((_B, _C), jnp.bfloat16),
            jax.ShapeDtypeStruct((_B, 1), jnp.float32),
            jax.ShapeDtypeStruct((_B, 1), jnp.float32),
            jax.ShapeDtypeStruct((1, 1), jnp.float32),
            jax.ShapeDtypeStruct((1, 1), jnp.float32),
        ],
    )(categorical, encodings)

    ent, entsum = pl.pallas_call(
        _entropy_kernel,
        grid=(_B // _BLK,),
        in_specs=[
            pl.BlockSpec((_BLK, _E), lambda i: (i, 0)),
            pl.BlockSpec((_E, _B), lambda i: (0, 0)),
            pl.BlockSpec((1, _B), lambda i: (0, 0)),
            pl.BlockSpec((_BLK, 1), lambda i: (i, 0)),
            pl.BlockSpec((_B, _C), lambda i: (0, 0)),
            pl.BlockSpec((_BLK, 1), lambda i: (i, 0)),
        ],
        out_specs=[
            pl.BlockSpec((_BLK, 1), lambda i: (i, 0)),
            pl.BlockSpec((1, 1), lambda i: (0, 0)),
        ],
        out_shape=[
            jax.ShapeDtypeStruct((_B, 1), jnp.float32),
            jax.ShapeDtypeStruct((1, 1), jnp.float32),
        ],
        compiler_params=pltpu.CompilerParams(
            dimension_semantics=("arbitrary",)),
    )(encodings, encodings.T, sq.T, sq, onehot, mg)

    neighbourhood_entropy = ent[:, 0]
    number_of_clusters = nclust[0, 0]
    average_confidence = conf_sum[0, 0] / _B
    average_neigh_entropy = entsum[0, 0] / _B
    loss = _LOSS_WEIGHT * average_neigh_entropy
    return (encodings, neighbourhood_entropy, number_of_clusters,
            average_confidence, average_neigh_entropy, loss)
